# baseline (speedup 1.0000x reference)
import dataclasses

import jax
import jax.numpy as jnp
from jax import lax
from jax.experimental import pallas as pl
from jax.experimental.pallas import tpu as pltpu
from jax.experimental.pallas import tpu_sc as plsc

DIM = 64
PAD_IDX = 0
SCALE = 8.0
NC = 2
NS = 16
L = 16
NW = NC * NS


def _make_body(n_rows_per_worker, chunk):
    n_chunks = n_rows_per_worker // chunk

    def body(table_hbm, idx_hbm, out_hbm, idx_v, rows_v, sem):
        wid = lax.axis_index("c") * NS + lax.axis_index("s")
        w_base = wid * n_rows_per_worker

        @pl.loop(0, n_chunks)
        def _(i):
            base = w_base + i * chunk
            pltpu.sync_copy(idx_hbm.at[pl.ds(base, chunk)], idx_v)
            pltpu.async_copy(table_hbm.at[idx_v], rows_v, sem).wait()

            @pl.loop(0, chunk)
            def _(r):
                row = rows_v.at[r]
                for c in range(DIM // L):
                    sl = pl.ds(c * L, L)
                    row[sl] = row[sl] * SCALE

            @pl.loop(0, chunk // L)
            def _(g):
                idx_reg = idx_v[pl.ds(g * L, L)]
                m = idx_reg == PAD_IDX

                @pl.when(jnp.any(m))
                def _():
                    r_idx = g * L + lax.iota(jnp.int32, L)
                    zeros = jnp.zeros((L,), jnp.float32)
                    for e in range(DIM):
                        c_idx = jnp.full((L,), e, jnp.int32)
                        plsc.store_scatter(rows_v, [r_idx, c_idx], zeros,
                                           mask=m)

            pltpu.sync_copy(rows_v, out_hbm.at[pl.ds(base, chunk)])

    return body


@jax.jit
def _embed(idx_flat, table):
    n = idx_flat.shape[0]
    rows_per_worker = n // NW
    chunk = 128
    mesh = plsc.VectorSubcoreMesh(core_axis_name="c", subcore_axis_name="s")
    cp = pltpu.CompilerParams(needs_layout_passes=False,
                              use_tc_tiling_on_sc=False)
    kern = pl.kernel(
        _make_body(rows_per_worker, chunk),
        out_type=jax.ShapeDtypeStruct((n, DIM), jnp.float32),
        mesh=mesh,
        scratch_types=[
            pltpu.VMEM((chunk,), jnp.int32),
            pltpu.VMEM((chunk, DIM), jnp.float32),
            pltpu.SemaphoreType.DMA,
        ],
        compiler_params=cp,
    )
    return kern(table, idx_flat)


def kernel(token_tensor, table):
    idx_flat = token_tensor.reshape(-1).astype(jnp.int32)
    out = _embed(idx_flat, table)
    return out.reshape(token_tensor.shape + (DIM,))


# speedup vs baseline: 1.2301x; 1.2301x over previous
import jax
import jax.numpy as jnp
from jax import lax
from jax.experimental import pallas as pl
from jax.experimental.pallas import tpu as pltpu
from jax.experimental.pallas import tpu_sc as plsc

DIM = 64
PAD_IDX = 0
SCALE = 8.0
NC = 2
NS = 16
L = 16
NW = NC * NS
G = 128
K = 4
GROUP = K * G


def _make_body(rows_per_worker):
    n_groups = rows_per_worker // GROUP
    chunks_per_worker = rows_per_worker // G
    assert n_groups * GROUP == rows_per_worker
    assert n_groups % 2 == 0 and n_groups >= 4

    def body(table_hbm, idx_hbm, out_hbm, idx_v, rows_v, sem_g0, sem_g1,
             sem_w0, sem_w1):
        wid = lax.axis_index("c") * NS + lax.axis_index("s")
        w_base = wid * rows_per_worker
        sem_g = [sem_g0, sem_g1]
        sem_w = [sem_w0, sem_w1]

        pltpu.sync_copy(idx_hbm.at[pl.ds(wid * chunks_per_worker,
                                         chunks_per_worker)], idx_v)

        def issue_gathers(g, s):
            for k in range(K):
                pltpu.async_copy(
                    table_hbm.at[idx_v.at[g * K + k]],
                    rows_v.at[s].at[pl.ds(k * G, G)],
                    sem_g[s])

        def wait_gathers(s):
            pltpu.make_async_copy(out_hbm.at[pl.ds(0, GROUP)],
                                  rows_v.at[s], sem_g[s]).wait()

        def wait_writeback(g, s):
            pltpu.make_async_copy(
                rows_v.at[s],
                out_hbm.at[pl.ds(w_base + g * GROUP, GROUP)],
                sem_w[s]).wait()

        def compute(g, s):
            rows_set = rows_v.at[s]

            @pl.loop(0, GROUP)
            def _(r):
                row = rows_set.at[r]
                for c in range(DIM // L):
                    sl = pl.ds(c * L, L)
                    row[sl] = row[sl] * SCALE

            for k in range(K):
                irow = idx_v.at[g * K + k]

                @pl.loop(0, G // L)
                def _(q):
                    idx_reg = irow[pl.ds(q * L, L)]
                    m = idx_reg == PAD_IDX

                    @pl.when(jnp.any(m))
                    def _():
                        r_idx = k * G + q * L + lax.iota(jnp.int32, L)
                        zeros = jnp.zeros((L,), jnp.float32)
                        for e in range(DIM):
                            c_idx = jnp.full((L,), e, jnp.int32)
                            plsc.store_scatter(rows_set, [r_idx, c_idx],
                                               zeros, mask=m)

        issue_gathers(0, 0)

        @pl.loop(0, n_groups // 2)
        def _(gp):
            for s in range(2):
                g = gp * 2 + s
                ns = 1 - s

                @pl.when(g + 1 < n_groups)
                def _():
                    @pl.when(g >= 1)
                    def _():
                        wait_writeback(g - 1, ns)
                    issue_gathers(g + 1, ns)

                wait_gathers(s)
                compute(g, s)
                pltpu.async_copy(
                    rows_v.at[s],
                    out_hbm.at[pl.ds(w_base + g * GROUP, GROUP)],
                    sem_w[s])

        wait_writeback(n_groups - 2, 0)
        wait_writeback(n_groups - 1, 1)

    return body


@jax.jit
def _embed(idx_2d, table):
    n = idx_2d.shape[0] * idx_2d.shape[1]
    rows_per_worker = n // NW
    mesh = plsc.VectorSubcoreMesh(core_axis_name="c", subcore_axis_name="s")
    cp = pltpu.CompilerParams(needs_layout_passes=False,
                              use_tc_tiling_on_sc=False)
    kern = pl.kernel(
        _make_body(rows_per_worker),
        out_type=jax.ShapeDtypeStruct((n, DIM), jnp.float32),
        mesh=mesh,
        scratch_types=[
            pltpu.VMEM((rows_per_worker // G, G), jnp.int32),
            pltpu.VMEM((2, GROUP, DIM), jnp.float32),
            pltpu.SemaphoreType.DMA,
            pltpu.SemaphoreType.DMA,
            pltpu.SemaphoreType.DMA,
            pltpu.SemaphoreType.DMA,
        ],
        compiler_params=cp,
    )
    return kern(table, idx_2d)


def kernel(token_tensor, table):
    idx_flat = token_tensor.reshape(-1).astype(jnp.int32)
    out = _embed(idx_flat.reshape(-1, G), table)
    return out.reshape(token_tensor.shape + (DIM,))
